# baseline (speedup 1.0000x reference)
import functools

import jax
import jax.numpy as jnp
from jax import lax
from jax.experimental import pallas as pl
from jax.experimental.pallas import tpu as pltpu

E = 8
K = 2
B = 16
L = 512
NC = 5
FLAT = 1024 * (L // 8)


def _router_body(x_ref, w1_ref, b1_ref, wr_ref, br_ref,
                 h_ref, ss_ref, se_ref, sg_ref, cv2_ref, st_ref, en_ref):
    x = x_ref[...]
    zc = jnp.zeros((B, 1), jnp.float32)
    xl = jnp.concatenate([zc, x[:, :-1]], axis=1)
    xr = jnp.concatenate([x[:, 1:], zc], axis=1)
    pooled_cols = []
    for co in range(16):
        h_co = (w1_ref[co, 0] * xl + w1_ref[co, 1] * x + w1_ref[co, 2] * xr
                + b1_ref[0, co])
        h_co = jnp.maximum(h_co, 0.0)
        h_ref[:, co, :] = h_co
        pooled_cols.append(jnp.mean(h_co, axis=1, keepdims=True))
    pooled = jnp.concatenate(pooled_cols, axis=1)

    logits = lax.dot_general(pooled, wr_ref[...],
                             (((1,), (1,)), ((), ())),
                             preferred_element_type=jnp.float32)
    logits = logits + br_ref[...]
    m = jnp.max(logits, axis=1, keepdims=True)
    ex = jnp.exp(logits - m)
    probs = ex / jnp.sum(ex, axis=1, keepdims=True)

    eio = lax.broadcasted_iota(jnp.int32, (B, E), 1)
    g0 = jnp.max(probs, axis=1, keepdims=True)
    i0 = jnp.min(jnp.where(probs == g0, eio, E + 1), axis=1, keepdims=True)
    probs2 = jnp.where(eio == i0, -1.0, probs)
    g1 = jnp.max(probs2, axis=1, keepdims=True)
    i1 = jnp.min(jnp.where(probs2 == g1, eio, E + 1), axis=1, keepdims=True)
    gsum = g0 + g1
    g0n = g0 / gsum
    g1n = g1 / gsum

    mp = jnp.mean(probs, axis=0, keepdims=True)
    mu = jnp.mean(mp, axis=1, keepdims=True)
    var = jnp.sum((mp - mu) ** 2, axis=1, keepdims=True) / (E - 1)
    cv2_ref[...] = var / (mu + 1e-10) ** 2

    sio = lax.broadcasted_iota(jnp.int32, (B, 1), 0)
    key0 = (i0 * (B * K) + K * sio).astype(jnp.float32)
    key1 = (i1 * (B * K) + K * sio + 1).astype(jnp.float32)
    eyef = (lax.broadcasted_iota(jnp.int32, (B, B), 0)
            == lax.broadcasted_iota(jnp.int32, (B, B), 1)).astype(jnp.float32)

    def _row(col):
        return lax.dot_general(col, eyef, (((0,), (0,)), ((), ())),
                               preferred_element_type=jnp.float32)

    k0r, k1r = _row(key0), _row(key1)
    pos0 = (jnp.sum((k0r < key0).astype(jnp.float32), axis=1, keepdims=True)
            + jnp.sum((k1r < key0).astype(jnp.float32), axis=1,
                      keepdims=True))
    pos1 = (jnp.sum((k0r < key1).astype(jnp.float32), axis=1, keepdims=True)
            + jnp.sum((k1r < key1).astype(jnp.float32), axis=1,
                      keepdims=True))
    pio = lax.broadcasted_iota(jnp.int32, (B, B * K), 1)
    oh0 = (pos0.astype(jnp.int32) == pio).astype(jnp.float32)
    oh1 = (pos1.astype(jnp.int32) == pio).astype(jnp.float32)

    def _scatter(v0, v1):
        return (jnp.sum(oh0 * v0, axis=0, keepdims=True)
                + jnp.sum(oh1 * v1, axis=0, keepdims=True))

    siof = sio.astype(jnp.float32)
    ss_ref[...] = _scatter(siof, siof).astype(jnp.int32)
    se_ref[...] = _scatter(i0.astype(jnp.float32),
                           i1.astype(jnp.float32)).astype(jnp.int32)
    sg_ref[...] = _scatter(g0n, g1n)

    eio1 = lax.broadcasted_iota(jnp.int32, (1, E), 1)
    stv = (jnp.sum((i0 < eio1).astype(jnp.float32), axis=0, keepdims=True)
           + jnp.sum((i1 < eio1).astype(jnp.float32), axis=0, keepdims=True))
    env = (jnp.sum((i0 <= eio1).astype(jnp.float32), axis=0, keepdims=True)
           + jnp.sum((i1 <= eio1).astype(jnp.float32), axis=0,
                     keepdims=True))
    st_ref[...] = stv.astype(jnp.int32)
    en_ref[...] = env.astype(jnp.int32)


def _router(x2d, w1, b1, wr, br):
    return pl.pallas_call(
        _router_body,
        out_shape=(
            jax.ShapeDtypeStruct((B, 16, L), jnp.float32),
            jax.ShapeDtypeStruct((1, B * K), jnp.int32),
            jax.ShapeDtypeStruct((1, B * K), jnp.int32),
            jax.ShapeDtypeStruct((1, B * K), jnp.float32),
            jax.ShapeDtypeStruct((1, 1), jnp.float32),
            jax.ShapeDtypeStruct((1, E), jnp.int32),
            jax.ShapeDtypeStruct((1, E), jnp.int32),
        ),
    )(x2d, w1, b1, wr, br)


def _conv(h, w_ref, b_ref, seg):
    ci, ll = h.shape
    pos = lax.broadcasted_iota(jnp.int32, (1, ll), 1) % seg
    zc = jnp.zeros((ci, 1), jnp.float32)
    hl = jnp.concatenate([zc, h[:, :-1]], axis=1)
    hl = jnp.where(pos == 0, 0.0, hl)
    hr = jnp.concatenate([h[:, 1:], zc], axis=1)
    hr = jnp.where(pos == seg - 1, 0.0, hr)
    w = w_ref[...]
    acc = lax.dot_general(w[0, 0], hl, (((1,), (0,)), ((), ())),
                          preferred_element_type=jnp.float32)
    acc += lax.dot_general(w[0, 1], h, (((1,), (0,)), ((), ())),
                           preferred_element_type=jnp.float32)
    acc += lax.dot_general(w[0, 2], hr, (((1,), (0,)), ((), ())),
                           preferred_element_type=jnp.float32)
    return acc + b_ref[0]


def _pool2(h):
    co, ll = h.shape
    zc = jnp.zeros((co, 1), jnp.float32)
    hs = jnp.concatenate([h[:, 1:], zc], axis=1)
    hm = jnp.maximum(h, hs)
    ii = lax.broadcasted_iota(jnp.int32, (ll, ll // 2), 0)
    jj = lax.broadcasted_iota(jnp.int32, (ll, ll // 2), 1)
    sel = (ii == 2 * jj).astype(jnp.float32)
    return lax.dot_general(hm, sel, (((1,), (0,)), ((), ())),
                           preferred_element_type=jnp.float32)


CAP = 4
NR = B // CAP


def _chain(h, refs):
    (w1, b1, w2, b2, w3, b3, w4, b4, w5, b5, w6, b6) = refs
    h = _conv(h, w1, b1, L)
    h = jnp.maximum(_conv(h, w2, b2, L), 0.0)
    h = _pool2(h)
    h = _conv(h, w3, b3, L // 2)
    h = jnp.maximum(_conv(h, w4, b4, L // 2), 0.0)
    h = _pool2(h)
    h = _conv(h, w5, b5, L // 4)
    h = jnp.maximum(_conv(h, w6, b6, L // 4), 0.0)
    h = _pool2(h)
    return h


def _expert_body(ss_ref, st_ref, en_ref, *rest):
    hrefs = rest[0:CAP]
    wrefs = rest[CAP:CAP + 12]
    out_ref = rest[CAP + 12]
    ebuf = rest[CAP + 13]
    sem = rest[CAP + 14]
    i = pl.program_id(0)
    e = i // NR
    base = st_ref[e] + CAP * (i % NR)
    act = en_ref[e] - base

    @pl.when(act > 0)
    def _():
        hcat = jnp.concatenate([hr[0] for hr in hrefs], axis=1)
        res = _chain(hcat, wrefs)
        for c in range(CAP):
            ebuf[c] = res[:, c * (L // 8):(c + 1) * (L // 8)]
        for c in range(CAP):
            @pl.when(act > c)
            def _(c=c):
                cp = pltpu.make_async_copy(
                    ebuf.at[c], out_ref.at[base + c], sem)
                cp.start()
                cp.wait()


def _experts(hstem, ss, st, en, wts, bss):
    chans = [(16, 32), (32, 64), (64, 128), (128, 256), (256, 512),
             (512, 1024)]

    def _hmap(c):
        def f(i, ss_, st_, en_):
            j = jnp.clip(st_[i // NR] + CAP * (i % NR) + c, 0, B * K - 1)
            return (ss_[j], 0, 0)
        return f

    in_specs = [pl.BlockSpec((1, 16, L), _hmap(c)) for c in range(CAP)]
    args = [hstem] * CAP
    for (ci, co), w, b in zip(chans, wts, bss):
        in_specs.append(pl.BlockSpec(
            (1, 3, co, ci), lambda i, *_: (i // NR, 0, 0, 0)))
        in_specs.append(pl.BlockSpec(
            (1, co, 1), lambda i, *_: (i // NR, 0, 0)))
        args.append(w)
        args.append(b)
    grid_spec = pltpu.PrefetchScalarGridSpec(
        num_scalar_prefetch=3,
        grid=(E * NR,),
        in_specs=in_specs,
        out_specs=pl.BlockSpec(memory_space=pl.ANY),
        scratch_shapes=[
            pltpu.VMEM((CAP, 1024, L // 8), jnp.float32),
            pltpu.SemaphoreType.DMA,
        ],
    )
    return pl.pallas_call(
        _expert_body,
        grid_spec=grid_spec,
        out_shape=jax.ShapeDtypeStruct((B * K, 1024, L // 8), jnp.float32),
    )(ss, st, en, *args)


def _head_body(ss_ref, sg_ref, eo_ref, wfc1_ref, bfc1_ref,
               wfc2_ref, bfc2_ref, wfc3_ref, bfc3_ref, out_ref, acc):
    k = pl.program_id(0)
    nk = pl.num_programs(0)

    @pl.when(k == 0)
    def _():
        acc[...] = jnp.zeros_like(acc)

    sio = lax.broadcasted_iota(jnp.int32, (B, B * K), 0)
    mcomb = jnp.where(ss_ref[...] == sio, sg_ref[...], 0.0)
    comb = lax.dot_general(mcomb, eo_ref[...], (((1,), (0,)), ((), ())),
                           preferred_element_type=jnp.float32)
    acc[...] += lax.dot_general(comb, wfc1_ref[...],
                                (((1,), (1,)), ((), ())),
                                preferred_element_type=jnp.float32)

    @pl.when(k == nk - 1)
    def _():
        z = jnp.maximum(acc[...] + bfc1_ref[...], 0.0)
        z = lax.dot_general(z, wfc2_ref[...], (((1,), (1,)), ((), ())),
                            preferred_element_type=jnp.float32)
        z = jnp.maximum(z + bfc2_ref[...], 0.0)
        z = lax.dot_general(z, wfc3_ref[...], (((1,), (1,)), ((), ())),
                            preferred_element_type=jnp.float32)
        out_ref[...] = z + bfc3_ref[...]


def _head(eo2d, ss, sg, wfc1, bfc1, wfc2, bfc2, wfc3, bfc3):
    nk = 8
    ck = FLAT // nk
    return pl.pallas_call(
        _head_body,
        grid=(nk,),
        in_specs=[
            pl.BlockSpec((1, B * K), lambda k: (0, 0)),
            pl.BlockSpec((1, B * K), lambda k: (0, 0)),
            pl.BlockSpec((B * K, ck), lambda k: (0, k)),
            pl.BlockSpec((256, ck), lambda k: (0, k)),
            pl.BlockSpec((1, 256), lambda k: (0, 0)),
            pl.BlockSpec((64, 256), lambda k: (0, 0)),
            pl.BlockSpec((1, 64), lambda k: (0, 0)),
            pl.BlockSpec((NC, 64), lambda k: (0, 0)),
            pl.BlockSpec((1, NC), lambda k: (0, 0)),
        ],
        out_specs=pl.BlockSpec((B, NC), lambda k: (0, 0)),
        out_shape=jax.ShapeDtypeStruct((B, NC), jnp.float32),
        scratch_shapes=[pltpu.VMEM((B, 256), jnp.float32)],
    )(ss, sg, eo2d, wfc1, bfc1, wfc2, bfc2, wfc3, bfc3)


@jax.jit
def kernel(x, Wconv1, bconv1, Wr, br, Wc1, bc1, Wc2, bc2, Wc3, bc3,
           Wc4, bc4, Wc5, bc5, Wc6, bc6, Wfc1, bfc1, Wfc2, bfc2,
           Wfc3, bfc3):
    x2d = x.reshape(B, L)
    w1 = Wconv1.reshape(16, 3)
    b1 = bconv1.reshape(1, 16)
    brr = br.reshape(1, E)

    hstem, ss, se, sg, cv2, st, en = _router(x2d, w1, b1, Wr, brr)

    wts = [jnp.transpose(w, (0, 3, 1, 2))
           for w in (Wc1, Wc2, Wc3, Wc4, Wc5, Wc6)]
    bss = [b[..., None] for b in (bc1, bc2, bc3, bc4, bc5, bc6)]
    del se
    eo = _experts(hstem, ss.reshape(B * K), st.reshape(E), en.reshape(E),
                  wts, bss)

    logits = _head(eo.reshape(B * K, FLAT), ss, sg,
                   Wfc1, bfc1.reshape(1, 256), Wfc2, bfc2.reshape(1, 64),
                   Wfc3, bfc3.reshape(1, NC))
    return (logits, cv2[0, 0])


# speedup vs baseline: 1.0714x; 1.0714x over previous
import functools

import jax
import jax.numpy as jnp
from jax import lax
from jax.experimental import pallas as pl
from jax.experimental.pallas import tpu as pltpu

E = 8
K = 2
B = 16
L = 512
NC = 5
FLAT = 1024 * (L // 8)


def _router_body(x_ref, w1_ref, b1_ref, wr_ref, br_ref,
                 h_ref, ss_ref, se_ref, sg_ref, cv2_ref, st_ref, en_ref):
    x = x_ref[...]
    zc = jnp.zeros((B, 1), jnp.float32)
    xl = jnp.concatenate([zc, x[:, :-1]], axis=1)
    xr = jnp.concatenate([x[:, 1:], zc], axis=1)
    pooled_cols = []
    for co in range(16):
        h_co = (w1_ref[co, 0] * xl + w1_ref[co, 1] * x + w1_ref[co, 2] * xr
                + b1_ref[0, co])
        h_co = jnp.maximum(h_co, 0.0)
        h_ref[:, co, :] = h_co
        pooled_cols.append(jnp.mean(h_co, axis=1, keepdims=True))
    pooled = jnp.concatenate(pooled_cols, axis=1)

    logits = lax.dot_general(pooled, wr_ref[...],
                             (((1,), (1,)), ((), ())),
                             preferred_element_type=jnp.float32)
    logits = logits + br_ref[...]
    m = jnp.max(logits, axis=1, keepdims=True)
    ex = jnp.exp(logits - m)
    probs = ex / jnp.sum(ex, axis=1, keepdims=True)

    eio = lax.broadcasted_iota(jnp.int32, (B, E), 1)
    g0 = jnp.max(probs, axis=1, keepdims=True)
    i0 = jnp.min(jnp.where(probs == g0, eio, E + 1), axis=1, keepdims=True)
    probs2 = jnp.where(eio == i0, -1.0, probs)
    g1 = jnp.max(probs2, axis=1, keepdims=True)
    i1 = jnp.min(jnp.where(probs2 == g1, eio, E + 1), axis=1, keepdims=True)
    gsum = g0 + g1
    g0n = g0 / gsum
    g1n = g1 / gsum

    mp = jnp.mean(probs, axis=0, keepdims=True)
    mu = jnp.mean(mp, axis=1, keepdims=True)
    var = jnp.sum((mp - mu) ** 2, axis=1, keepdims=True) / (E - 1)
    cv2_ref[...] = var / (mu + 1e-10) ** 2

    sio = lax.broadcasted_iota(jnp.int32, (B, 1), 0)
    key0 = (i0 * (B * K) + K * sio).astype(jnp.float32)
    key1 = (i1 * (B * K) + K * sio + 1).astype(jnp.float32)
    eyef = (lax.broadcasted_iota(jnp.int32, (B, B), 0)
            == lax.broadcasted_iota(jnp.int32, (B, B), 1)).astype(jnp.float32)

    def _row(col):
        return lax.dot_general(col, eyef, (((0,), (0,)), ((), ())),
                               preferred_element_type=jnp.float32)

    k0r, k1r = _row(key0), _row(key1)
    pos0 = (jnp.sum((k0r < key0).astype(jnp.float32), axis=1, keepdims=True)
            + jnp.sum((k1r < key0).astype(jnp.float32), axis=1,
                      keepdims=True))
    pos1 = (jnp.sum((k0r < key1).astype(jnp.float32), axis=1, keepdims=True)
            + jnp.sum((k1r < key1).astype(jnp.float32), axis=1,
                      keepdims=True))
    pio = lax.broadcasted_iota(jnp.int32, (B, B * K), 1)
    oh0 = (pos0.astype(jnp.int32) == pio).astype(jnp.float32)
    oh1 = (pos1.astype(jnp.int32) == pio).astype(jnp.float32)

    def _scatter(v0, v1):
        return (jnp.sum(oh0 * v0, axis=0, keepdims=True)
                + jnp.sum(oh1 * v1, axis=0, keepdims=True))

    siof = sio.astype(jnp.float32)
    ss_ref[...] = _scatter(siof, siof).astype(jnp.int32)
    se_ref[...] = _scatter(i0.astype(jnp.float32),
                           i1.astype(jnp.float32)).astype(jnp.int32)
    sg_ref[...] = _scatter(g0n, g1n)

    eio1 = lax.broadcasted_iota(jnp.int32, (1, E), 1)
    stv = (jnp.sum((i0 < eio1).astype(jnp.float32), axis=0, keepdims=True)
           + jnp.sum((i1 < eio1).astype(jnp.float32), axis=0, keepdims=True))
    env = (jnp.sum((i0 <= eio1).astype(jnp.float32), axis=0, keepdims=True)
           + jnp.sum((i1 <= eio1).astype(jnp.float32), axis=0,
                     keepdims=True))
    st_ref[...] = stv.astype(jnp.int32)
    en_ref[...] = env.astype(jnp.int32)


def _router(x2d, w1, b1, wr, br):
    return pl.pallas_call(
        _router_body,
        out_shape=(
            jax.ShapeDtypeStruct((B, 16, L), jnp.float32),
            jax.ShapeDtypeStruct((1, B * K), jnp.int32),
            jax.ShapeDtypeStruct((1, B * K), jnp.int32),
            jax.ShapeDtypeStruct((1, B * K), jnp.float32),
            jax.ShapeDtypeStruct((1, 1), jnp.float32),
            jax.ShapeDtypeStruct((1, E), jnp.int32),
            jax.ShapeDtypeStruct((1, E), jnp.int32),
        ),
    )(x2d, w1, b1, wr, br)


def _conv(h, w_ref, b_ref, seg):
    ci, ll = h.shape
    pos = lax.broadcasted_iota(jnp.int32, (1, ll), 1) % seg
    zc = jnp.zeros((ci, 1), jnp.float32)
    hl = jnp.concatenate([zc, h[:, :-1]], axis=1)
    hl = jnp.where(pos == 0, 0.0, hl)
    hr = jnp.concatenate([h[:, 1:], zc], axis=1)
    hr = jnp.where(pos == seg - 1, 0.0, hr)
    w = w_ref[...]
    acc = lax.dot_general(w[0, 0], hl, (((1,), (0,)), ((), ())),
                          preferred_element_type=jnp.float32)
    acc += lax.dot_general(w[0, 1], h, (((1,), (0,)), ((), ())),
                           preferred_element_type=jnp.float32)
    acc += lax.dot_general(w[0, 2], hr, (((1,), (0,)), ((), ())),
                           preferred_element_type=jnp.float32)
    return acc + b_ref[0]


def _pool2(h):
    co, ll = h.shape
    zc = jnp.zeros((co, 1), jnp.float32)
    hs = jnp.concatenate([h[:, 1:], zc], axis=1)
    hm = jnp.maximum(h, hs)
    ii = lax.broadcasted_iota(jnp.int32, (ll, ll // 2), 0)
    jj = lax.broadcasted_iota(jnp.int32, (ll, ll // 2), 1)
    sel = (ii == 2 * jj).astype(jnp.float32)
    return lax.dot_general(hm, sel, (((1,), (0,)), ((), ())),
                           preferred_element_type=jnp.float32)


CAP = 4
NR = B // CAP


def _chain(h, refs):
    (w1, b1, w2, b2, w3, b3, w4, b4, w5, b5, w6, b6) = refs
    h = _conv(h, w1, b1, L)
    h = jnp.maximum(_conv(h, w2, b2, L), 0.0)
    h = _pool2(h)
    h = _conv(h, w3, b3, L // 2)
    h = jnp.maximum(_conv(h, w4, b4, L // 2), 0.0)
    h = _pool2(h)
    h = _conv(h, w5, b5, L // 4)
    h = jnp.maximum(_conv(h, w6, b6, L // 4), 0.0)
    h = _pool2(h)
    return h


def _expert_body(ss_ref, st_ref, en_ref, h_ref,
                 w1, b1, w2, b2, w3, b3, w4, b4, w5, b5, w6, b6,
                 out_ref, ebuf, sem):
    wrefs = (w1, b1, w2, b2, w3, b3, w4, b4, w5, b5, w6, b6)
    e = pl.program_id(0)
    base = st_ref[e]
    end = en_ref[e]
    nit = (end - base + CAP - 1) // CAP

    def it_body(t, carry):
        b0 = base + CAP * t
        cols = []
        for c in range(CAP):
            j = jnp.clip(b0 + c, 0, B * K - 1)
            cols.append(h_ref[ss_ref[j]])
        hcat = jnp.concatenate(cols, axis=1)
        res = _chain(hcat, wrefs)
        for c in range(CAP):
            ebuf[c] = res[:, c * (L // 8):(c + 1) * (L // 8)]
        for c in range(CAP):
            @pl.when(b0 + c < end)
            def _(c=c):
                cp = pltpu.make_async_copy(
                    ebuf.at[c], out_ref.at[b0 + c], sem)
                cp.start()
                cp.wait()
        return carry

    lax.fori_loop(0, nit, it_body, 0)


def _experts(hstem, ss, st, en, wts, bss):
    chans = [(16, 32), (32, 64), (64, 128), (128, 256), (256, 512),
             (512, 1024)]
    in_specs = [pl.BlockSpec((B, 16, L), lambda e, *_: (0, 0, 0))]
    args = [hstem]
    for (ci, co), w, b in zip(chans, wts, bss):
        in_specs.append(pl.BlockSpec(
            (1, 3, co, ci), lambda e, *_: (e, 0, 0, 0)))
        in_specs.append(pl.BlockSpec(
            (1, co, 1), lambda e, *_: (e, 0, 0)))
        args.append(w)
        args.append(b)
    grid_spec = pltpu.PrefetchScalarGridSpec(
        num_scalar_prefetch=3,
        grid=(E,),
        in_specs=in_specs,
        out_specs=pl.BlockSpec(memory_space=pl.ANY),
        scratch_shapes=[
            pltpu.VMEM((CAP, 1024, L // 8), jnp.float32),
            pltpu.SemaphoreType.DMA,
        ],
    )
    return pl.pallas_call(
        _expert_body,
        grid_spec=grid_spec,
        out_shape=jax.ShapeDtypeStruct((B * K, 1024, L // 8), jnp.float32),
    )(ss, st, en, *args)


def _head_body(ss_ref, sg_ref, eo_ref, wfc1_ref, bfc1_ref,
               wfc2_ref, bfc2_ref, wfc3_ref, bfc3_ref, out_ref, acc):
    k = pl.program_id(0)
    nk = pl.num_programs(0)

    @pl.when(k == 0)
    def _():
        acc[...] = jnp.zeros_like(acc)

    sio = lax.broadcasted_iota(jnp.int32, (B, B * K), 0)
    mcomb = jnp.where(ss_ref[...] == sio, sg_ref[...], 0.0)
    comb = lax.dot_general(mcomb, eo_ref[...], (((1,), (0,)), ((), ())),
                           preferred_element_type=jnp.float32)
    acc[...] += lax.dot_general(comb, wfc1_ref[...],
                                (((1,), (1,)), ((), ())),
                                preferred_element_type=jnp.float32)

    @pl.when(k == nk - 1)
    def _():
        z = jnp.maximum(acc[...] + bfc1_ref[...], 0.0)
        z = lax.dot_general(z, wfc2_ref[...], (((1,), (1,)), ((), ())),
                            preferred_element_type=jnp.float32)
        z = jnp.maximum(z + bfc2_ref[...], 0.0)
        z = lax.dot_general(z, wfc3_ref[...], (((1,), (1,)), ((), ())),
                            preferred_element_type=jnp.float32)
        out_ref[...] = z + bfc3_ref[...]


def _head(eo2d, ss, sg, wfc1, bfc1, wfc2, bfc2, wfc3, bfc3):
    nk = 8
    ck = FLAT // nk
    return pl.pallas_call(
        _head_body,
        grid=(nk,),
        in_specs=[
            pl.BlockSpec((1, B * K), lambda k: (0, 0)),
            pl.BlockSpec((1, B * K), lambda k: (0, 0)),
            pl.BlockSpec((B * K, ck), lambda k: (0, k)),
            pl.BlockSpec((256, ck), lambda k: (0, k)),
            pl.BlockSpec((1, 256), lambda k: (0, 0)),
            pl.BlockSpec((64, 256), lambda k: (0, 0)),
            pl.BlockSpec((1, 64), lambda k: (0, 0)),
            pl.BlockSpec((NC, 64), lambda k: (0, 0)),
            pl.BlockSpec((1, NC), lambda k: (0, 0)),
        ],
        out_specs=pl.BlockSpec((B, NC), lambda k: (0, 0)),
        out_shape=jax.ShapeDtypeStruct((B, NC), jnp.float32),
        scratch_shapes=[pltpu.VMEM((B, 256), jnp.float32)],
    )(ss, sg, eo2d, wfc1, bfc1, wfc2, bfc2, wfc3, bfc3)


@jax.jit
def kernel(x, Wconv1, bconv1, Wr, br, Wc1, bc1, Wc2, bc2, Wc3, bc3,
           Wc4, bc4, Wc5, bc5, Wc6, bc6, Wfc1, bfc1, Wfc2, bfc2,
           Wfc3, bfc3):
    x2d = x.reshape(B, L)
    w1 = Wconv1.reshape(16, 3)
    b1 = bconv1.reshape(1, 16)
    brr = br.reshape(1, E)

    hstem, ss, se, sg, cv2, st, en = _router(x2d, w1, b1, Wr, brr)

    wts = [jnp.transpose(w, (0, 3, 1, 2))
           for w in (Wc1, Wc2, Wc3, Wc4, Wc5, Wc6)]
    bss = [b[..., None] for b in (bc1, bc2, bc3, bc4, bc5, bc6)]
    del se
    eo = _experts(hstem, ss.reshape(B * K), st.reshape(E), en.reshape(E),
                  wts, bss)

    logits = _head(eo.reshape(B * K, FLAT), ss, sg,
                   Wfc1, bfc1.reshape(1, 256), Wfc2, bfc2.reshape(1, 64),
                   Wfc3, bfc3.reshape(1, NC))
    return (logits, cv2[0, 0])


# speedup vs baseline: 1.0726x; 1.0012x over previous
import functools

import jax
import jax.numpy as jnp
from jax import lax
from jax.experimental import pallas as pl
from jax.experimental.pallas import tpu as pltpu

E = 8
K = 2
B = 16
L = 512
NC = 5
FLAT = 1024 * (L // 8)


def _router_body(x_ref, w1_ref, b1_ref, wr_ref, br_ref,
                 h_ref, ss_ref, se_ref, sg_ref, cv2_ref, st_ref, en_ref):
    x = x_ref[...]
    zc = jnp.zeros((B, 1), jnp.float32)
    xl = jnp.concatenate([zc, x[:, :-1]], axis=1)
    xr = jnp.concatenate([x[:, 1:], zc], axis=1)
    pooled_cols = []
    for co in range(16):
        h_co = (w1_ref[co, 0] * xl + w1_ref[co, 1] * x + w1_ref[co, 2] * xr
                + b1_ref[0, co])
        h_co = jnp.maximum(h_co, 0.0)
        h_ref[:, co, :] = h_co
        pooled_cols.append(jnp.mean(h_co, axis=1, keepdims=True))
    pooled = jnp.concatenate(pooled_cols, axis=1)

    logits = lax.dot_general(pooled, wr_ref[...],
                             (((1,), (1,)), ((), ())),
                             preferred_element_type=jnp.float32)
    logits = logits + br_ref[...]
    m = jnp.max(logits, axis=1, keepdims=True)
    ex = jnp.exp(logits - m)
    probs = ex / jnp.sum(ex, axis=1, keepdims=True)

    eio = lax.broadcasted_iota(jnp.int32, (B, E), 1)
    g0 = jnp.max(probs, axis=1, keepdims=True)
    i0 = jnp.min(jnp.where(probs == g0, eio, E + 1), axis=1, keepdims=True)
    probs2 = jnp.where(eio == i0, -1.0, probs)
    g1 = jnp.max(probs2, axis=1, keepdims=True)
    i1 = jnp.min(jnp.where(probs2 == g1, eio, E + 1), axis=1, keepdims=True)
    gsum = g0 + g1
    g0n = g0 / gsum
    g1n = g1 / gsum

    mp = jnp.mean(probs, axis=0, keepdims=True)
    mu = jnp.mean(mp, axis=1, keepdims=True)
    var = jnp.sum((mp - mu) ** 2, axis=1, keepdims=True) / (E - 1)
    cv2_ref[...] = var / (mu + 1e-10) ** 2

    sio = lax.broadcasted_iota(jnp.int32, (B, 1), 0)
    key0 = (i0 * (B * K) + K * sio).astype(jnp.float32)
    key1 = (i1 * (B * K) + K * sio + 1).astype(jnp.float32)
    eyef = (lax.broadcasted_iota(jnp.int32, (B, B), 0)
            == lax.broadcasted_iota(jnp.int32, (B, B), 1)).astype(jnp.float32)

    def _row(col):
        return lax.dot_general(col, eyef, (((0,), (0,)), ((), ())),
                               preferred_element_type=jnp.float32)

    k0r, k1r = _row(key0), _row(key1)
    pos0 = (jnp.sum((k0r < key0).astype(jnp.float32), axis=1, keepdims=True)
            + jnp.sum((k1r < key0).astype(jnp.float32), axis=1,
                      keepdims=True))
    pos1 = (jnp.sum((k0r < key1).astype(jnp.float32), axis=1, keepdims=True)
            + jnp.sum((k1r < key1).astype(jnp.float32), axis=1,
                      keepdims=True))
    pio = lax.broadcasted_iota(jnp.int32, (B, B * K), 1)
    oh0 = (pos0.astype(jnp.int32) == pio).astype(jnp.float32)
    oh1 = (pos1.astype(jnp.int32) == pio).astype(jnp.float32)

    def _scatter(v0, v1):
        return (jnp.sum(oh0 * v0, axis=0, keepdims=True)
                + jnp.sum(oh1 * v1, axis=0, keepdims=True))

    siof = sio.astype(jnp.float32)
    ss_ref[...] = _scatter(siof, siof).astype(jnp.int32)
    se_ref[...] = _scatter(i0.astype(jnp.float32),
                           i1.astype(jnp.float32)).astype(jnp.int32)
    sg_ref[...] = _scatter(g0n, g1n)

    eio1 = lax.broadcasted_iota(jnp.int32, (1, E), 1)
    stv = (jnp.sum((i0 < eio1).astype(jnp.float32), axis=0, keepdims=True)
           + jnp.sum((i1 < eio1).astype(jnp.float32), axis=0, keepdims=True))
    env = (jnp.sum((i0 <= eio1).astype(jnp.float32), axis=0, keepdims=True)
           + jnp.sum((i1 <= eio1).astype(jnp.float32), axis=0,
                     keepdims=True))
    st_ref[...] = stv.astype(jnp.int32)
    en_ref[...] = env.astype(jnp.int32)


def _router(x2d, w1, b1, wr, br):
    return pl.pallas_call(
        _router_body,
        out_shape=(
            jax.ShapeDtypeStruct((B, 16, L), jnp.float32),
            jax.ShapeDtypeStruct((1, B * K), jnp.int32),
            jax.ShapeDtypeStruct((1, B * K), jnp.int32),
            jax.ShapeDtypeStruct((1, B * K), jnp.float32),
            jax.ShapeDtypeStruct((1, 1), jnp.float32),
            jax.ShapeDtypeStruct((1, E), jnp.int32),
            jax.ShapeDtypeStruct((1, E), jnp.int32),
        ),
    )(x2d, w1, b1, wr, br)


def _conv(h, w_ref, b_ref, seg, dt=jnp.float32):
    ci, ll = h.shape
    pos = lax.broadcasted_iota(jnp.int32, (1, ll), 1) % seg
    zc = jnp.zeros((ci, 1), jnp.float32)
    hl = jnp.concatenate([zc, h[:, :-1]], axis=1)
    hl = jnp.where(pos == 0, 0.0, hl)
    hr = jnp.concatenate([h[:, 1:], zc], axis=1)
    hr = jnp.where(pos == seg - 1, 0.0, hr)
    w = w_ref[...].astype(dt)
    hl = hl.astype(dt)
    hc = h.astype(dt)
    hr = hr.astype(dt)
    acc = lax.dot_general(w[0, 0], hl, (((1,), (0,)), ((), ())),
                          preferred_element_type=jnp.float32)
    acc += lax.dot_general(w[0, 1], hc, (((1,), (0,)), ((), ())),
                           preferred_element_type=jnp.float32)
    acc += lax.dot_general(w[0, 2], hr, (((1,), (0,)), ((), ())),
                           preferred_element_type=jnp.float32)
    return acc + b_ref[0]


def _pool2(h):
    co, ll = h.shape
    zc = jnp.zeros((co, 1), jnp.float32)
    hs = jnp.concatenate([h[:, 1:], zc], axis=1)
    hm = jnp.maximum(h, hs)
    ii = lax.broadcasted_iota(jnp.int32, (ll, ll // 2), 0)
    jj = lax.broadcasted_iota(jnp.int32, (ll, ll // 2), 1)
    sel = (ii == 2 * jj).astype(jnp.float32)
    return lax.dot_general(hm, sel, (((1,), (0,)), ((), ())),
                           preferred_element_type=jnp.float32)


CAP = 4
NR = B // CAP


def _chain(h, refs):
    (w1, b1, w2, b2, w3, b3, w4, b4, w5, b5, w6, b6) = refs
    h = _conv(h, w1, b1, L)
    h = jnp.maximum(_conv(h, w2, b2, L), 0.0)
    h = _pool2(h)
    h = _conv(h, w3, b3, L // 2)
    h = jnp.maximum(_conv(h, w4, b4, L // 2), 0.0)
    h = _pool2(h)
    h = _conv(h, w5, b5, L // 4)
    h = jnp.maximum(_conv(h, w6, b6, L // 4), 0.0)
    h = _pool2(h)
    return h


def _expert_body(ss_ref, st_ref, en_ref, h_ref,
                 w1, b1, w2, b2, w3, b3, w4, b4, w5, b5, w6, b6,
                 out_ref, ebuf, sem):
    wrefs = (w1, b1, w2, b2, w3, b3, w4, b4, w5, b5, w6, b6)
    e = pl.program_id(0)
    base = st_ref[e]
    end = en_ref[e]
    nit = (end - base + CAP - 1) // CAP

    def it_body(t, carry):
        b0 = base + CAP * t
        cols = []
        for c in range(CAP):
            j = jnp.clip(b0 + c, 0, B * K - 1)
            cols.append(h_ref[ss_ref[j]])
        hcat = jnp.concatenate(cols, axis=1)
        res = _chain(hcat, wrefs)
        for c in range(CAP):
            ebuf[c] = res[:, c * (L // 8):(c + 1) * (L // 8)]
        for c in range(CAP):
            @pl.when(b0 + c < end)
            def _(c=c):
                cp = pltpu.make_async_copy(
                    ebuf.at[c], out_ref.at[b0 + c], sem)
                cp.start()
                cp.wait()
        return carry

    lax.fori_loop(0, nit, it_body, 0)


def _experts(hstem, ss, st, en, wts, bss):
    chans = [(16, 32), (32, 64), (64, 128), (128, 256), (256, 512),
             (512, 1024)]
    in_specs = [pl.BlockSpec((B, 16, L), lambda e, *_: (0, 0, 0))]
    args = [hstem]
    for (ci, co), w, b in zip(chans, wts, bss):
        in_specs.append(pl.BlockSpec(
            (1, 3, co, ci), lambda e, *_: (e, 0, 0, 0)))
        in_specs.append(pl.BlockSpec(
            (1, co, 1), lambda e, *_: (e, 0, 0)))
        args.append(w)
        args.append(b)
    grid_spec = pltpu.PrefetchScalarGridSpec(
        num_scalar_prefetch=3,
        grid=(E,),
        in_specs=in_specs,
        out_specs=pl.BlockSpec(memory_space=pl.ANY),
        scratch_shapes=[
            pltpu.VMEM((CAP, 1024, L // 8), jnp.float32),
            pltpu.SemaphoreType.DMA,
        ],
    )
    return pl.pallas_call(
        _expert_body,
        grid_spec=grid_spec,
        out_shape=jax.ShapeDtypeStruct((B * K, 1024, L // 8), jnp.float32),
    )(ss, st, en, *args)


def _head_body(ss_ref, sg_ref, eo_ref, wfc1_ref, bfc1_ref,
               wfc2_ref, bfc2_ref, wfc3_ref, bfc3_ref, out_ref, acc):
    k = pl.program_id(0)
    nk = pl.num_programs(0)

    @pl.when(k == 0)
    def _():
        acc[...] = jnp.zeros_like(acc)

    sio = lax.broadcasted_iota(jnp.int32, (B, B * K), 0)
    mcomb = jnp.where(ss_ref[...] == sio, sg_ref[...], 0.0)
    comb = lax.dot_general(mcomb, eo_ref[...], (((1,), (0,)), ((), ())),
                           preferred_element_type=jnp.float32)
    acc[...] += lax.dot_general(comb, wfc1_ref[...],
                                (((1,), (1,)), ((), ())),
                                preferred_element_type=jnp.float32)

    @pl.when(k == nk - 1)
    def _():
        z = jnp.maximum(acc[...] + bfc1_ref[...], 0.0)
        z = lax.dot_general(z, wfc2_ref[...], (((1,), (1,)), ((), ())),
                            preferred_element_type=jnp.float32)
        z = jnp.maximum(z + bfc2_ref[...], 0.0)
        z = lax.dot_general(z, wfc3_ref[...], (((1,), (1,)), ((), ())),
                            preferred_element_type=jnp.float32)
        out_ref[...] = z + bfc3_ref[...]


def _head(eo2d, ss, sg, wfc1, bfc1, wfc2, bfc2, wfc3, bfc3):
    nk = 8
    ck = FLAT // nk
    return pl.pallas_call(
        _head_body,
        grid=(nk,),
        in_specs=[
            pl.BlockSpec((1, B * K), lambda k: (0, 0)),
            pl.BlockSpec((1, B * K), lambda k: (0, 0)),
            pl.BlockSpec((B * K, ck), lambda k: (0, k)),
            pl.BlockSpec((256, ck), lambda k: (0, k)),
            pl.BlockSpec((1, 256), lambda k: (0, 0)),
            pl.BlockSpec((64, 256), lambda k: (0, 0)),
            pl.BlockSpec((1, 64), lambda k: (0, 0)),
            pl.BlockSpec((NC, 64), lambda k: (0, 0)),
            pl.BlockSpec((1, NC), lambda k: (0, 0)),
        ],
        out_specs=pl.BlockSpec((B, NC), lambda k: (0, 0)),
        out_shape=jax.ShapeDtypeStruct((B, NC), jnp.float32),
        scratch_shapes=[pltpu.VMEM((B, 256), jnp.float32)],
    )(ss, sg, eo2d, wfc1, bfc1, wfc2, bfc2, wfc3, bfc3)


@jax.jit
def kernel(x, Wconv1, bconv1, Wr, br, Wc1, bc1, Wc2, bc2, Wc3, bc3,
           Wc4, bc4, Wc5, bc5, Wc6, bc6, Wfc1, bfc1, Wfc2, bfc2,
           Wfc3, bfc3):
    x2d = x.reshape(B, L)
    w1 = Wconv1.reshape(16, 3)
    b1 = bconv1.reshape(1, 16)
    brr = br.reshape(1, E)

    hstem, ss, se, sg, cv2, st, en = _router(x2d, w1, b1, Wr, brr)

    wts = [jnp.transpose(w, (0, 3, 1, 2))
           for w in (Wc1, Wc2, Wc3, Wc4, Wc5, Wc6)]
    bss = [b[..., None] for b in (bc1, bc2, bc3, bc4, bc5, bc6)]
    del se
    eo = _experts(hstem, ss.reshape(B * K), st.reshape(E), en.reshape(E),
                  wts, bss)

    logits = _head(eo.reshape(B * K, FLAT), ss, sg,
                   Wfc1, bfc1.reshape(1, 256), Wfc2, bfc2.reshape(1, 64),
                   Wfc3, bfc3.reshape(1, NC))
    return (logits, cv2[0, 0])
